# baseline (speedup 1.0000x reference)
import functools

import jax
import jax.numpy as jnp
from jax import lax
from jax.experimental import pallas as pl
from jax.experimental.pallas import tpu as pltpu
from jax.experimental.pallas import tpu_sc as plsc

NUM_ENTITIES = 50000
NUM_RELATIONS = 1000
EMBED_DIM = 64
MAX_ACTIONS = 256
BATCH = 1024

NUM_WORKERS = 32
BPW = BATCH // NUM_WORKERS
HALF = 128
UNITS = BPW * 2


def _gather_spaces_body(head_hbm, rsp_hbm, esp_hbm, mask_hbm,
                        rspb_hbm, espb_hbm, mskb_hbm,
                        head_v, rsp_v, esp_v, msk_v, sem):
    cid = lax.axis_index("c")
    sid = lax.axis_index("s")
    wid = sid * 2 + cid
    base = wid * BPW

    pltpu.sync_copy(head_hbm.at[pl.ds(base, BPW)], head_v)
    c1 = pltpu.async_copy(rsp_hbm.at[head_v], rsp_v, sem)
    c2 = pltpu.async_copy(esp_hbm.at[head_v], esp_v, sem)
    c3 = pltpu.async_copy(mask_hbm.at[head_v], msk_v, sem)
    c1.wait()
    c2.wait()
    c3.wait()
    pltpu.sync_copy(rsp_v, rspb_hbm.at[pl.ds(base, BPW)])
    pltpu.sync_copy(esp_v, espb_hbm.at[pl.ds(base, BPW)])
    pltpu.sync_copy(msk_v, mskb_hbm.at[pl.ds(base, BPW)])


def _emb_body(ent_hbm, rel_hbm, rspb_hbm, espb_hbm, mskb_hbm, out_hbm,
              rsp_v, esp_v, msk_v, remb_v, eemb_v, out_v, gsem, wsem0, wsem1):
    cid = lax.axis_index("c")
    sid = lax.axis_index("s")
    wid = sid * 2 + cid
    base = wid * BPW

    pltpu.sync_copy(rspb_hbm.at[pl.ds(base, BPW)], rsp_v)
    pltpu.sync_copy(espb_hbm.at[pl.ds(base, BPW)], esp_v)
    pltpu.sync_copy(mskb_hbm.at[pl.ds(base, BPW)], msk_v)

    wsems = (wsem0, wsem1)

    def issue_gathers(i, h, b):
        pltpu.async_copy(rel_hbm.at[rsp_v.at[i, pl.ds(h * HALF, HALF)]],
                         remb_v.at[b], gsem)
        pltpu.async_copy(ent_hbm.at[esp_v.at[i, pl.ds(h * HALF, HALF)]],
                         eemb_v.at[b], gsem)

    def wait_gathers(b):
        pltpu.make_async_copy(rel_hbm.at[rsp_v.at[0, pl.ds(0, HALF)]],
                              remb_v.at[b], gsem).wait()
        pltpu.make_async_copy(ent_hbm.at[esp_v.at[0, pl.ds(0, HALF)]],
                              eemb_v.at[b], gsem).wait()

    def compute_unit(i, h, b):
        def grp_body(g, _):
            a0 = pl.multiple_of(h * HALF + g * 16, 16)
            mvec = msk_v[i, pl.ds(a0, 16)]
            for l in range(16):
                a = g * 16 + l
                mv = jnp.full((16,), mvec[l], dtype=jnp.float32)
                for c in range(4):
                    sl = pl.ds(c * 16, 16)
                    out_v[b, a, pl.ds(c * 16, 16)] = remb_v[b, a, sl] * mv
                    out_v[b, a, pl.ds(EMBED_DIM + c * 16, 16)] = (
                        eemb_v[b, a, sl] * mv)
            return _

        lax.fori_loop(0, HALF // 16, grp_body, None)

    def start_write(i, h, b):
        return pltpu.async_copy(
            out_v.at[b], out_hbm.at[base + i, pl.ds(h * HALF, HALF)],
            wsems[b])

    def wait_write(i, h, b):
        pltpu.make_async_copy(
            out_v.at[b], out_hbm.at[base + i, pl.ds(h * HALF, HALF)],
            wsems[b]).wait()

    issue_gathers(0, 0, 0)

    def step(t, carry):
        for b in range(2):
            u = 2 * t + b
            i = u // 2
            h = b
            wait_gathers(b)
            nxt = u + 1
            ni = nxt // 2
            nh = nxt % 2

            @pl.when(nxt < UNITS)
            def _():
                issue_gathers(ni, nh, 1 - b)

            @pl.when(u >= 2)
            def _():
                wait_write(i - 1, h, b)

            compute_unit(i, h, b)
            start_write(i, h, b)
        return carry

    lax.fori_loop(0, BPW, step, None)
    wait_write(BPW - 1, 0, 0)
    wait_write(BPW - 1, 1, 1)


@jax.jit
def _sc_call(entity_table, relation_table, action_mask, head,
             r_space, e_space):
    mesh = plsc.VectorSubcoreMesh(core_axis_name="c", subcore_axis_name="s")

    gather_spaces = pl.kernel(
        _gather_spaces_body,
        out_type=(
            jax.ShapeDtypeStruct((BATCH, MAX_ACTIONS), jnp.int32),
            jax.ShapeDtypeStruct((BATCH, MAX_ACTIONS), jnp.int32),
            jax.ShapeDtypeStruct((BATCH, MAX_ACTIONS), jnp.float32),
        ),
        mesh=mesh,
        scratch_types=[
            pltpu.VMEM((BPW,), jnp.int32),
            pltpu.VMEM((BPW, MAX_ACTIONS), jnp.int32),
            pltpu.VMEM((BPW, MAX_ACTIONS), jnp.int32),
            pltpu.VMEM((BPW, MAX_ACTIONS), jnp.float32),
            pltpu.SemaphoreType.DMA,
        ],
    )
    rsp_b, esp_b, msk_b = gather_spaces(head, r_space, e_space, action_mask)

    emb = pl.kernel(
        _emb_body,
        out_type=jax.ShapeDtypeStruct((BATCH, MAX_ACTIONS, 2 * EMBED_DIM),
                                      jnp.float32),
        mesh=mesh,
        compiler_params=pltpu.CompilerParams(use_tc_tiling_on_sc=False),
        scratch_types=[
            pltpu.VMEM((BPW, MAX_ACTIONS), jnp.int32),
            pltpu.VMEM((BPW, MAX_ACTIONS), jnp.int32),
            pltpu.VMEM((BPW, MAX_ACTIONS), jnp.float32),
            pltpu.VMEM((2, HALF, EMBED_DIM), jnp.float32),
            pltpu.VMEM((2, HALF, EMBED_DIM), jnp.float32),
            pltpu.VMEM((2, HALF, 2 * EMBED_DIM), jnp.float32),
            pltpu.SemaphoreType.DMA,
            pltpu.SemaphoreType.DMA,
            pltpu.SemaphoreType.DMA,
        ],
    )
    return emb(entity_table, relation_table, rsp_b, esp_b, msk_b)


def kernel(entity_table, relation_table, action_mask, head, r_space, e_space):
    head = head.astype(jnp.int32)
    return _sc_call(entity_table, relation_table, action_mask, head,
                    r_space, e_space)
